# baseline (speedup 1.0000x reference)
import functools

import jax
import jax.numpy as jnp
from jax import lax
from jax.experimental import pallas as pl
from jax.experimental.pallas import tpu as pltpu
from jax.experimental.pallas import tpu_sc as plsc

B = 4096
HIST = 200
D = 16

NC = 2
NS = 16
NW = NC * NS

SEQ_N = B * HIST
SEQ_PER_W = SEQ_N // NW
CH = 1280
NCH = SEQ_PER_W // CH
IPG = 128
GPC = CH // IPG
SMALL_PER_W = B // NW


def _split_columns(slab_v, ncols, nrows, outs):
  def body(k, _):
    rows = k * 16 + lax.broadcasted_iota(jnp.int32, (16,), 0)
    for c in range(ncols):
      cols = jnp.full((16,), c, jnp.int32)
      v = plsc.load_gather(slab_v, [rows, cols])
      outs[c][pl.ds(k * 16, 16)] = v
    return 0
  lax.fori_loop(0, nrows // 16, body, 0)


def _sc_body(seq2d, wseq0, wseq1, wseq2,
             tus, wu0, wu1, wu2, wu3, wu4,
             tis, wi0, wi1, wi2, tts,
             user_o, seq_o, tseq_o, tside_o,
             slab_v, i0_v, i1_v, i2_v, r0_v, r1_v, r2_v,
             sl5_v, sl3_v, sidx_v, sidx2_v, srows_v, sem):
  wid = lax.axis_index("s") * NC + lax.axis_index("c")
  base = wid * SEQ_PER_W

  wseq = [wseq0, wseq1, wseq2]
  icols = [i0_v, i1_v, i2_v]
  rows = [r0_v, r1_v, r2_v]

  def chunk_body(c, _):
    pos = base + c * CH
    pltpu.sync_copy(seq2d.at[pl.ds(pos, CH)], slab_v)
    _split_columns(slab_v, 3, CH, icols)

    def fire(j, _):
      for t in range(3):
        pltpu.async_copy(
            wseq[t].at[icols[t].at[pl.ds(j * IPG, IPG)]],
            rows[t].at[pl.ds(j * IPG, IPG)], sem)
      return 0
    lax.fori_loop(0, GPC, fire, 0)

    def drain(j, _):
      pltpu.make_async_copy(
          wseq[0].at[i0_v.at[pl.ds(0, IPG)]],
          r0_v.at[pl.ds(0, IPG)], sem).wait()
      return 0
    lax.fori_loop(0, 3 * GPC, drain, 0)

    for t in range(3):
      pltpu.sync_copy(rows[t], seq_o.at[pl.ds(pos, CH), pl.ds(16 * t, 16)])
    return 0

  lax.fori_loop(0, NCH, chunk_body, 0)

  sbase = wid * SMALL_PER_W
  scol = [sidx_v, i0_v, i1_v, i2_v, sidx2_v]

  def small_group(src2d, slab, ncols, tables, out):
    pltpu.sync_copy(src2d.at[pl.ds(sbase, SMALL_PER_W)], slab)
    cols = [scol[c].at[pl.ds(0, SMALL_PER_W)] for c in range(ncols)]
    _split_columns(slab, ncols, SMALL_PER_W, cols)
    for c in range(ncols):
      pltpu.async_copy(tables[c].at[cols[c]], srows_v, sem).wait()
      pltpu.sync_copy(
          srows_v,
          out.at[pl.ds(sbase, SMALL_PER_W), pl.ds(16 * c, 16)])

  small_group(tus, sl5_v, 5, [wu0, wu1, wu2, wu3, wu4], user_o)
  small_group(tts, sl3_v, 3, wseq, tseq_o)
  small_group(tis, sl3_v, 3, [wi0, wi1, wi2], tside_o)


_sc_call = functools.partial(
    pl.kernel,
    mesh=plsc.VectorSubcoreMesh(
        core_axis_name="c", subcore_axis_name="s", num_cores=NC),
    out_type=[
        jax.ShapeDtypeStruct((B, 5 * D), jnp.float32),
        jax.ShapeDtypeStruct((SEQ_N, 3 * D), jnp.float32),
        jax.ShapeDtypeStruct((B, 3 * D), jnp.float32),
        jax.ShapeDtypeStruct((B, 3 * D), jnp.float32),
    ],
    scratch_types=[
        pltpu.VMEM((CH, 3), jnp.int32),
        pltpu.VMEM((CH,), jnp.int32),
        pltpu.VMEM((CH,), jnp.int32),
        pltpu.VMEM((CH,), jnp.int32),
        pltpu.VMEM((CH, D), jnp.float32),
        pltpu.VMEM((CH, D), jnp.float32),
        pltpu.VMEM((CH, D), jnp.float32),
        pltpu.VMEM((SMALL_PER_W, 5), jnp.int32),
        pltpu.VMEM((SMALL_PER_W, 3), jnp.int32),
        pltpu.VMEM((SMALL_PER_W,), jnp.int32),
        pltpu.VMEM((SMALL_PER_W,), jnp.int32),
        pltpu.VMEM((SMALL_PER_W, D), jnp.float32),
        pltpu.SemaphoreType.DMA,
    ],
    compiler_params=pltpu.CompilerParams(
        use_tc_tiling_on_sc=False, needs_layout_passes=False),
)(_sc_body)


MB = 512


def _mask_body(x_ref, o_ref):
  x = x_ref[...]
  xf = (x != 0).astype(jnp.float32)
  r = lax.broadcasted_iota(jnp.int32, (3 * HIST, HIST), 0)
  c = lax.broadcasted_iota(jnp.int32, (3 * HIST, HIST), 1)
  sel = (r == 3 * c).astype(jnp.float32)
  y = jax.lax.dot(xf, sel, precision=jax.lax.Precision.HIGHEST)
  o_ref[...] = y > 0.5


_mask_call = pl.pallas_call(
    _mask_body,
    grid=(B // MB,),
    in_specs=[pl.BlockSpec((MB, 3 * HIST), lambda i: (i, 0))],
    out_specs=pl.BlockSpec((MB, HIST), lambda i: (i, 0)),
    out_shape=jax.ShapeDtypeStruct((B, HIST), jnp.bool_),
)


@jax.jit
def kernel(dense_inputs, target_user_side, seq_inputs, target_item_seq,
           target_item_side,
           W_seq0, W_seq1, W_seq2,
           W_user0, W_user1, W_user2, W_user3, W_user4,
           W_item0, W_item1, W_item2):
  del dense_inputs
  seq2d = seq_inputs.reshape(SEQ_N, 3)

  user_side, seq_embed, tseq, tside = _sc_call(
      seq2d, W_seq0, W_seq1, W_seq2,
      target_user_side, W_user0, W_user1, W_user2, W_user3, W_user4,
      target_item_side, W_item0, W_item1, W_item2, target_item_seq)

  mask_bool = _mask_call(seq_inputs.reshape(B, 3 * HIST))
  return (mask_bool, user_side, seq_embed.reshape(B, HIST, 3 * D),
          tseq, tside)


# speedup vs baseline: 1.0729x; 1.0729x over previous
import functools

import jax
import jax.numpy as jnp
from jax import lax
from jax.experimental import pallas as pl
from jax.experimental.pallas import tpu as pltpu
from jax.experimental.pallas import tpu_sc as plsc

B = 4096
HIST = 200
D = 16

NC = 2
NS = 16
NW = NC * NS

SEQ_N = B * HIST
SEQ_PER_W = SEQ_N // NW
CH = 1280
NCH = SEQ_PER_W // CH
NPAIR = NCH // 2
IPG = 128
GPC = CH // IPG
SMALL_PER_W = B // NW


def _split_columns(slab_v, ncols, nrows, outs, mask_v=None):
  def body(k, _):
    lane = lax.broadcasted_iota(jnp.int32, (16,), 0)
    flat0 = k * (16 * ncols)
    for c in range(ncols):
      v = plsc.load_gather(slab_v, [flat0 + ncols * lane + c])
      outs[c][pl.ds(k * 16, 16)] = v
      if c == 0 and mask_v is not None:
        mask_v[pl.ds(k * 16, 16)] = jnp.where(
            v != 0, jnp.int32(1), jnp.int32(0))
    return 0
  lax.fori_loop(0, nrows // 16, body, 0)


def _sc_body(seq_flat, wseq0, wseq1, wseq2,
             tus_flat, wu0, wu1, wu2, wu3, wu4,
             tis_flat, wi0, wi1, wi2, tts_flat,
             mask_o, user_o, seq_o, tseq_o, tside_o,
             slab0, slab1, i0a, i1a, i2a, i0b, i1b, i2b,
             m0, m1, rows0, rows1,
             sl5_v, sl3_v, c0_v, c1_v, c2_v, c3_v, c4_v, srows_v,
             gsem, wsem0, wsem1, ssem):
  wid = lax.axis_index("s") * NC + lax.axis_index("c")
  base = wid * SEQ_PER_W

  wseq = [wseq0, wseq1, wseq2]
  slabs = [slab0, slab1]
  icols = [[i0a, i1a, i2a], [i0b, i1b, i2b]]
  masks = [m0, m1]
  rows = [rows0, rows1]
  wsems = [wsem0, wsem1]

  def load_extract(c, s):
    pos = base + c * CH
    pltpu.sync_copy(seq_flat.at[pl.ds(pos * 3, CH * 3)], slabs[s])
    _split_columns(slabs[s], 3, CH, icols[s], masks[s])
    pltpu.sync_copy(masks[s], mask_o.at[pl.ds(pos, CH)])

  def fire(t, s, p):
    def go(j, _):
      pltpu.async_copy(
          wseq[t].at[icols[s][t].at[pl.ds(j * IPG, IPG)]],
          rows[p].at[pl.ds(j * IPG, IPG)], gsem)
      return 0
    lax.fori_loop(0, GPC, go, 0)

  def drain():
    def go(j, _):
      pltpu.make_async_copy(
          wseq[0].at[i0a.at[pl.ds(0, IPG)]],
          rows0.at[pl.ds(0, IPG)], gsem).wait()
      return 0
    lax.fori_loop(0, GPC, go, 0)

  def wait_write(p):
    pltpu.make_async_copy(
        rows[p], seq_o.at[pl.ds(0, CH), pl.ds(0, 16)], wsems[p]).wait()

  def write(c, t, p):
    pos = base + c * CH
    pltpu.async_copy(
        rows[p], seq_o.at[pl.ds(pos, CH), pl.ds(16 * t, 16)], wsems[p])

  load_extract(0, 0)

  def pair_body(k, _):
    ca = 2 * k
    cb = 2 * k + 1
    for half, c in ((0, ca), (1, cb)):
      for t in range(3):
        stage = 3 * half + t
        p = stage % 2
        if stage < 2:
          @pl.when(k > 0)
          def _():
            wait_write(p)
        else:
          wait_write(p)
        fire(t, half, p)
        if stage == 0:
          load_extract(cb, 1)
        elif stage == 3:
          @pl.when(k < NPAIR - 1)
          def _():
            load_extract(ca + 2, 0)
        drain()
        write(c, t, p)
    return 0

  lax.fori_loop(0, NPAIR, pair_body, 0)
  wait_write(0)
  wait_write(1)

  sbase = wid * SMALL_PER_W
  scol = [c0_v, c1_v, c2_v, c3_v, c4_v]

  def small_group(src_flat, slab, ncols, tables, out):
    pltpu.sync_copy(
        src_flat.at[pl.ds(sbase * ncols, SMALL_PER_W * ncols)], slab)
    _split_columns(slab, ncols, SMALL_PER_W, scol)
    for c in range(ncols):
      pltpu.async_copy(tables[c].at[scol[c]], srows_v, ssem).wait()
      pltpu.sync_copy(
          srows_v,
          out.at[pl.ds(sbase, SMALL_PER_W), pl.ds(16 * c, 16)])

  small_group(tus_flat, sl5_v, 5, [wu0, wu1, wu2, wu3, wu4], user_o)
  small_group(tts_flat, sl3_v, 3, wseq, tseq_o)
  small_group(tis_flat, sl3_v, 3, [wi0, wi1, wi2], tside_o)


_sc_call = functools.partial(
    pl.kernel,
    mesh=plsc.VectorSubcoreMesh(
        core_axis_name="c", subcore_axis_name="s", num_cores=NC),
    out_type=[
        jax.ShapeDtypeStruct((SEQ_N,), jnp.int32),
        jax.ShapeDtypeStruct((B, 5 * D), jnp.float32),
        jax.ShapeDtypeStruct((SEQ_N, 3 * D), jnp.float32),
        jax.ShapeDtypeStruct((B, 3 * D), jnp.float32),
        jax.ShapeDtypeStruct((B, 3 * D), jnp.float32),
    ],
    scratch_types=[
        pltpu.VMEM((CH * 3,), jnp.int32),
        pltpu.VMEM((CH * 3,), jnp.int32),
        pltpu.VMEM((CH,), jnp.int32),
        pltpu.VMEM((CH,), jnp.int32),
        pltpu.VMEM((CH,), jnp.int32),
        pltpu.VMEM((CH,), jnp.int32),
        pltpu.VMEM((CH,), jnp.int32),
        pltpu.VMEM((CH,), jnp.int32),
        pltpu.VMEM((CH,), jnp.int32),
        pltpu.VMEM((CH,), jnp.int32),
        pltpu.VMEM((CH, D), jnp.float32),
        pltpu.VMEM((CH, D), jnp.float32),
        pltpu.VMEM((SMALL_PER_W * 5,), jnp.int32),
        pltpu.VMEM((SMALL_PER_W * 3,), jnp.int32),
        pltpu.VMEM((SMALL_PER_W,), jnp.int32),
        pltpu.VMEM((SMALL_PER_W,), jnp.int32),
        pltpu.VMEM((SMALL_PER_W,), jnp.int32),
        pltpu.VMEM((SMALL_PER_W,), jnp.int32),
        pltpu.VMEM((SMALL_PER_W,), jnp.int32),
        pltpu.VMEM((SMALL_PER_W, D), jnp.float32),
        pltpu.SemaphoreType.DMA,
        pltpu.SemaphoreType.DMA,
        pltpu.SemaphoreType.DMA,
        pltpu.SemaphoreType.DMA,
    ],
    compiler_params=pltpu.CompilerParams(
        use_tc_tiling_on_sc=False, needs_layout_passes=False),
)(_sc_body)


@jax.jit
def kernel(dense_inputs, target_user_side, seq_inputs, target_item_seq,
           target_item_side,
           W_seq0, W_seq1, W_seq2,
           W_user0, W_user1, W_user2, W_user3, W_user4,
           W_item0, W_item1, W_item2):
  del dense_inputs

  mask_i, user_side, seq_embed, tseq, tside = _sc_call(
      seq_inputs.reshape(-1), W_seq0, W_seq1, W_seq2,
      target_user_side.reshape(-1),
      W_user0, W_user1, W_user2, W_user3, W_user4,
      target_item_side.reshape(-1), W_item0, W_item1, W_item2,
      target_item_seq.reshape(-1))

  mask_bool = mask_i.reshape(B, HIST).astype(jnp.bool_)
  return (mask_bool, user_side, seq_embed.reshape(B, HIST, 3 * D),
          tseq, tside)


# speedup vs baseline: 2.3560x; 2.1959x over previous
import functools

import jax
import jax.numpy as jnp
from jax import lax
from jax.experimental import pallas as pl
from jax.experimental.pallas import tpu as pltpu
from jax.experimental.pallas import tpu_sc as plsc

B = 4096
HIST = 200
D = 16

NC = 2
NS = 16
NW = NC * NS

SEQ_N = B * HIST
SEQ_PER_W = SEQ_N // NW
CH = 1280
NCH = SEQ_PER_W // CH
NPAIR = NCH // 2
IPG = 128
GPC = CH // IPG
SMALL_PER_W = B // NW


def _sc_body(s0, s1, s2, wseq0, wseq1, wseq2,
             u0, u1, u2, u3, u4, wu0, wu1, wu2, wu3, wu4,
             ts0, ts1, ts2,
             ti0, ti1, ti2, wi0, wi1, wi2,
             mask_o, user_o, seq_o, tseq_o, tside_o,
             idx0, idx1, m0, m1, rows0, rows1,
             sidx_v, srows_v,
             gsem, wsem0, wsem1, ssem):
  wid = lax.axis_index("s") * NC + lax.axis_index("c")
  base = wid * SEQ_PER_W

  sidx = [s0, s1, s2]
  wseq = [wseq0, wseq1, wseq2]
  idxs = [idx0, idx1]
  masks = [m0, m1]
  rows = [rows0, rows1]
  wsems = [wsem0, wsem1]

  def fire(t, p):
    def go(j, _):
      pltpu.async_copy(
          wseq[t].at[idxs[p].at[pl.ds(j * IPG, IPG)]],
          rows[p].at[pl.ds(j * IPG, IPG)], gsem)
      return 0
    lax.fori_loop(0, GPC, go, 0)

  def drain():
    def go(j, _):
      pltpu.make_async_copy(
          wseq[0].at[idx0.at[pl.ds(0, IPG)]],
          rows0.at[pl.ds(0, IPG)], gsem).wait()
      return 0
    lax.fori_loop(0, GPC, go, 0)

  def wait_write(p):
    pltpu.make_async_copy(
        rows[p], seq_o.at[pl.ds(0, CH), pl.ds(0, 16)], wsems[p]).wait()

  def pair_body(k, _):
    for half in range(2):
      c = 2 * k + half
      pos = base + c * CH
      for t in range(3):
        stage = 3 * half + t
        p = stage % 2
        pltpu.sync_copy(sidx[t].at[pl.ds(pos, CH)], idxs[p])
        if stage < 2:
          @pl.when(k > 0)
          def _():
            wait_write(p)
        else:
          wait_write(p)
        fire(t, p)
        if t == 0:
          def mask_body(j, _):
            v = idxs[p][pl.ds(j * 16, 16)]
            masks[p][pl.ds(j * 16, 16)] = jnp.where(
                v != 0, jnp.int32(1), jnp.int32(0))
            return 0
          lax.fori_loop(0, CH // 16, mask_body, 0)
          pltpu.sync_copy(masks[p], mask_o.at[pl.ds(pos, CH)])
        drain()
        pltpu.async_copy(
            rows[p], seq_o.at[pl.ds(pos, CH), pl.ds(16 * t, 16)], wsems[p])
    return 0

  lax.fori_loop(0, NPAIR, pair_body, 0)
  wait_write(0)
  wait_write(1)

  sbase = wid * SMALL_PER_W
  small = (
      [(([u0, u1, u2, u3, u4])[i], ([wu0, wu1, wu2, wu3, wu4])[i], user_o, i)
       for i in range(5)]
      + [(([ts0, ts1, ts2])[i], wseq[i], tseq_o, i) for i in range(3)]
      + [(([ti0, ti1, ti2])[i], ([wi0, wi1, wi2])[i], tside_o, i)
         for i in range(3)]
  )
  for idx_hbm, table, out, col in small:
    pltpu.sync_copy(idx_hbm.at[pl.ds(sbase, SMALL_PER_W)], sidx_v)
    pltpu.async_copy(table.at[sidx_v], srows_v, ssem).wait()
    pltpu.sync_copy(
        srows_v, out.at[pl.ds(sbase, SMALL_PER_W), pl.ds(16 * col, 16)])


_sc_call = functools.partial(
    pl.kernel,
    mesh=plsc.VectorSubcoreMesh(
        core_axis_name="c", subcore_axis_name="s", num_cores=NC),
    out_type=[
        jax.ShapeDtypeStruct((SEQ_N,), jnp.int32),
        jax.ShapeDtypeStruct((B, 5 * D), jnp.float32),
        jax.ShapeDtypeStruct((SEQ_N, 3 * D), jnp.float32),
        jax.ShapeDtypeStruct((B, 3 * D), jnp.float32),
        jax.ShapeDtypeStruct((B, 3 * D), jnp.float32),
    ],
    scratch_types=[
        pltpu.VMEM((CH,), jnp.int32),
        pltpu.VMEM((CH,), jnp.int32),
        pltpu.VMEM((CH,), jnp.int32),
        pltpu.VMEM((CH,), jnp.int32),
        pltpu.VMEM((CH, D), jnp.float32),
        pltpu.VMEM((CH, D), jnp.float32),
        pltpu.VMEM((SMALL_PER_W,), jnp.int32),
        pltpu.VMEM((SMALL_PER_W, D), jnp.float32),
        pltpu.SemaphoreType.DMA,
        pltpu.SemaphoreType.DMA,
        pltpu.SemaphoreType.DMA,
        pltpu.SemaphoreType.DMA,
    ],
    compiler_params=pltpu.CompilerParams(
        use_tc_tiling_on_sc=False, needs_layout_passes=False),
)(_sc_body)


@jax.jit
def kernel(dense_inputs, target_user_side, seq_inputs, target_item_seq,
           target_item_side,
           W_seq0, W_seq1, W_seq2,
           W_user0, W_user1, W_user2, W_user3, W_user4,
           W_item0, W_item1, W_item2):
  del dense_inputs
  s = [seq_inputs[:, :, i].reshape(-1) for i in range(3)]
  u = [target_user_side[:, i] for i in range(5)]
  ts = [target_item_seq[:, i] for i in range(3)]
  ti = [target_item_side[:, i] for i in range(3)]

  mask_i, user_side, seq_embed, tseq, tside = _sc_call(
      s[0], s[1], s[2], W_seq0, W_seq1, W_seq2,
      u[0], u[1], u[2], u[3], u[4],
      W_user0, W_user1, W_user2, W_user3, W_user4,
      ts[0], ts[1], ts[2],
      ti[0], ti[1], ti[2], W_item0, W_item1, W_item2)

  mask_bool = mask_i.reshape(B, HIST).astype(jnp.bool_)
  return (mask_bool, user_side, seq_embed.reshape(B, HIST, 3 * D),
          tseq, tside)
